# baseline (speedup 1.0000x reference)
import functools

import jax
import jax.numpy as jnp
from jax.experimental import pallas as pl

VOCAB = 100000
BATCH = 32
Q_LEN = 8
VB = 16384


def _body(ls_ref, ids_ref, out_ref):
    j = pl.program_id(0)
    alpha = jax.nn.sigmoid(ls_ref[...])
    base = (1.0 - alpha) * jnp.float32(1.0 / VOCAB)
    log_base = jnp.maximum(jnp.log(base), jnp.float32(-1e6))
    log_peak = jnp.maximum(jnp.log(base + alpha), jnp.float32(-1e6))
    col = jax.lax.broadcasted_iota(jnp.int32, (BATCH, Q_LEN, VB), 2) + j * VB
    mask = col == ids_ref[...][..., None]
    out_ref[...] = jnp.where(mask, log_peak[..., None], log_base[..., None])


@jax.jit
def kernel(log_snr, input_ids):
    grid = (pl.cdiv(VOCAB, VB),)
    return pl.pallas_call(
        _body,
        grid=grid,
        in_specs=[
            pl.BlockSpec((BATCH, Q_LEN), lambda j: (0, 0)),
            pl.BlockSpec((BATCH, Q_LEN), lambda j: (0, 0)),
        ],
        out_specs=pl.BlockSpec((BATCH, Q_LEN, VB), lambda j: (0, 0, j)),
        out_shape=jax.ShapeDtypeStruct((BATCH, Q_LEN, VOCAB), jnp.float32),
    )(log_snr, input_ids.astype(jnp.int32))


# speedup vs baseline: 1.0077x; 1.0077x over previous
import functools

import jax
import jax.numpy as jnp
from jax.experimental import pallas as pl

VOCAB = 100000
BATCH = 32
Q_LEN = 8
BB = 4


def _body(ls_ref, ids_ref, out_ref):
    i = pl.program_id(0)
    alpha = jax.nn.sigmoid(ls_ref[pl.ds(i * BB, BB), :])
    base = (1.0 - alpha) * jnp.float32(1.0 / VOCAB)
    log_base = jnp.maximum(jnp.log(base), jnp.float32(-1e6))
    log_peak = jnp.maximum(jnp.log(base + alpha), jnp.float32(-1e6))
    col = jax.lax.broadcasted_iota(jnp.int32, (BB, Q_LEN, VOCAB), 2)
    mask = col == ids_ref[pl.ds(i * BB, BB), :][..., None]
    out_ref[...] = jnp.where(mask, log_peak[..., None], log_base[..., None])


@jax.jit
def kernel(log_snr, input_ids):
    grid = (BATCH // BB,)
    return pl.pallas_call(
        _body,
        grid=grid,
        in_specs=[
            pl.BlockSpec((BATCH, Q_LEN), lambda i: (0, 0)),
            pl.BlockSpec((BATCH, Q_LEN), lambda i: (0, 0)),
        ],
        out_specs=pl.BlockSpec((BB, Q_LEN, VOCAB), lambda i: (i, 0, 0)),
        out_shape=jax.ShapeDtypeStruct((BATCH, Q_LEN, VOCAB), jnp.float32),
    )(log_snr, input_ids.astype(jnp.int32))


# speedup vs baseline: 1.0422x; 1.0342x over previous
import functools

import jax
import jax.numpy as jnp
from jax.experimental import pallas as pl

VOCAB = 100000
BATCH = 32
Q_LEN = 8
BB = 2


def _body(ls_ref, ids_ref, out_ref):
    i = pl.program_id(0)
    alpha = jax.nn.sigmoid(ls_ref[pl.ds(i * BB, BB), :])
    base = (1.0 - alpha) * jnp.float32(1.0 / VOCAB)
    log_base = jnp.maximum(jnp.log(base), jnp.float32(-1e6))
    log_peak = jnp.maximum(jnp.log(base + alpha), jnp.float32(-1e6))
    col = jax.lax.broadcasted_iota(jnp.int32, (BB, Q_LEN, VOCAB), 2)
    mask = col == ids_ref[pl.ds(i * BB, BB), :][..., None]
    out_ref[...] = jnp.where(mask, log_peak[..., None], log_base[..., None])


@jax.jit
def kernel(log_snr, input_ids):
    grid = (BATCH // BB,)
    return pl.pallas_call(
        _body,
        grid=grid,
        in_specs=[
            pl.BlockSpec((BATCH, Q_LEN), lambda i: (0, 0)),
            pl.BlockSpec((BATCH, Q_LEN), lambda i: (0, 0)),
        ],
        out_specs=pl.BlockSpec((BB, Q_LEN, VOCAB), lambda i: (i, 0, 0)),
        out_shape=jax.ShapeDtypeStruct((BATCH, Q_LEN, VOCAB), jnp.float32),
    )(log_snr, input_ids.astype(jnp.int32))
